# Initial kernel scaffold; baseline (speedup 1.0000x reference)
import jax
import jax.numpy as jnp
from jax.experimental import pallas as pl


def kernel(x, edge_index, W_lin, b_lin, W_gate, b_gate):
    raise NotImplementedError("write your pallas kernel here")


# speedup vs baseline: 4.5070x; 4.5070x over previous
import functools

import jax
import jax.numpy as jnp
from jax import lax
from jax.experimental import pallas as pl
from jax.experimental.pallas import tpu as pltpu
from jax.experimental.pallas import tpu_sc as plsc

N = 10000
E = 160000
IN_DIM = 256
GRAPH_DIM = 256
NUM_GRAPH = 4

BLK = 2048
NB = (N + BLK - 1) // BLK

NC, NS, L = 2, 16, 16
NW = NC * NS
EPT = ((E // NW) + L - 1) // L * L
EPAD = EPT * NW
GROUPS = EPT // L


def _tc_body(x_ref, wl_ref, bl_ref, g8_ref, h_ref, pt_ref):
    h = jnp.dot(x_ref[...], wl_ref[...], preferred_element_type=jnp.float32)
    h = h + bl_ref[...]
    h_ref[...] = h
    pt_ref[...] = lax.dot_general(
        g8_ref[...], h, (((1,), (1,)), ((), ())),
        preferred_element_type=jnp.float32)


def _tc_project(x, W_lin, b_lin, g8):
    return pl.pallas_call(
        _tc_body,
        grid=(NB,),
        in_specs=[
            pl.BlockSpec((BLK, IN_DIM), lambda i: (i, 0)),
            pl.BlockSpec((IN_DIM, GRAPH_DIM), lambda i: (0, 0)),
            pl.BlockSpec((1, GRAPH_DIM), lambda i: (0, 0)),
            pl.BlockSpec((2 * NUM_GRAPH, GRAPH_DIM), lambda i: (0, 0)),
        ],
        out_specs=[
            pl.BlockSpec((BLK, GRAPH_DIM), lambda i: (i, 0)),
            pl.BlockSpec((2 * NUM_GRAPH, BLK), lambda i: (0, i)),
        ],
        out_shape=[
            jax.ShapeDtypeStruct((N, GRAPH_DIM), jnp.float32),
            jax.ShapeDtypeStruct((2 * NUM_GRAPH, N), jnp.float32),
        ],
    )(x, W_lin, b_lin, g8)


def _sc_body(pt_hbm, dst_hbm, src_hbm, bias_hbm, out_hbm,
             pt_v, dst_v, src_v, out_v, bias_v):
    wid = lax.axis_index("s") * NC + lax.axis_index("c")
    base = wid * EPT
    pltpu.sync_copy(pt_hbm, pt_v)
    pltpu.sync_copy(dst_hbm.at[pl.ds(base, EPT)], dst_v)
    pltpu.sync_copy(src_hbm.at[pl.ds(base, EPT)], src_v)
    pltpu.sync_copy(bias_hbm, bias_v)

    biases = [bias_v[k, :] for k in range(NUM_GRAPH)]
    lanes = lax.iota(jnp.int32, L)

    def group(g, _):
        d = dst_v[pl.ds(g * L, L)]
        s = src_v[pl.ds(g * L, L)]
        obase = g * (L * NUM_GRAPH) + lanes * NUM_GRAPH
        for k in range(NUM_GRAPH):
            a = plsc.load_gather(pt_v, [d + (k * N)])
            b = plsc.load_gather(pt_v, [s + ((NUM_GRAPH + k) * N)])
            t = a + b + biases[k]
            f = 1.0 / (1.0 + jnp.exp(-t))
            plsc.store_scatter(out_v, [obase + k], f)
        return 0

    lax.fori_loop(0, GROUPS, group, 0)
    pltpu.sync_copy(out_v, out_hbm.at[pl.ds(base * NUM_GRAPH, EPT * NUM_GRAPH)])


@functools.partial(
    pl.kernel,
    mesh=plsc.VectorSubcoreMesh(core_axis_name="c", subcore_axis_name="s"),
    out_type=jax.ShapeDtypeStruct((EPAD * NUM_GRAPH,), jnp.float32),
    compiler_params=pltpu.CompilerParams(needs_layout_passes=False),
    scratch_types=[
        pltpu.VMEM((2 * NUM_GRAPH * N,), jnp.float32),
        pltpu.VMEM((EPT,), jnp.int32),
        pltpu.VMEM((EPT,), jnp.int32),
        pltpu.VMEM((EPT * NUM_GRAPH,), jnp.float32),
        pltpu.VMEM((NUM_GRAPH, L), jnp.float32),
    ],
)
def _sc_gate(pt_hbm, dst_hbm, src_hbm, bias_hbm, out_hbm,
             pt_v, dst_v, src_v, out_v, bias_v):
    _sc_body(pt_hbm, dst_hbm, src_hbm, bias_hbm, out_hbm,
             pt_v, dst_v, src_v, out_v, bias_v)


def kernel(x, edge_index, W_lin, b_lin, W_gate, b_gate):
    g8 = jnp.concatenate([W_gate[:, :GRAPH_DIM], W_gate[:, GRAPH_DIM:]], axis=0)
    bl = b_lin.reshape(1, GRAPH_DIM)
    bias_b = jnp.broadcast_to(b_gate[:, None], (NUM_GRAPH, L))

    src = edge_index[0]
    dst = edge_index[1]
    pad = EPAD - E
    dst_p = jnp.pad(dst, (0, pad))
    src_p = jnp.pad(src, (0, pad))

    h, pt = _tc_project(x, W_lin, bl, g8)
    out = _sc_gate(pt.reshape(-1), dst_p, src_p, bias_b)
    factors = out.reshape(EPAD, NUM_GRAPH)[:E]
    return h, factors


# speedup vs baseline: 10.8301x; 2.4029x over previous
import functools

import jax
import jax.numpy as jnp
from jax import lax
from jax.experimental import pallas as pl
from jax.experimental.pallas import tpu as pltpu
from jax.experimental.pallas import tpu_sc as plsc

N = 10000
E = 160000
IN_DIM = 256
GRAPH_DIM = 256
NUM_GRAPH = 4

BLK = 2048
NB = (N + BLK - 1) // BLK

NC, NS, L = 2, 16, 16
NW = NC * NS
EB = 128
EPT = 5120
EPAD = EPT * NW
GROUPS = EPT // L


def _tc_body(x_ref, wl_ref, bl_ref, g8_ref, h_ref, pt_ref):
    h = jnp.dot(x_ref[...], wl_ref[...], preferred_element_type=jnp.float32)
    h = h + bl_ref[...]
    h_ref[...] = h
    pt_ref[...] = lax.dot_general(
        g8_ref[...], h, (((1,), (1,)), ((), ())),
        preferred_element_type=jnp.float32)


def _tc_project(x, W_lin, b_lin, g8):
    return pl.pallas_call(
        _tc_body,
        grid=(NB,),
        in_specs=[
            pl.BlockSpec((BLK, IN_DIM), lambda i: (i, 0)),
            pl.BlockSpec((IN_DIM, GRAPH_DIM), lambda i: (0, 0)),
            pl.BlockSpec((1, GRAPH_DIM), lambda i: (0, 0)),
            pl.BlockSpec((2 * NUM_GRAPH, GRAPH_DIM), lambda i: (0, 0)),
        ],
        out_specs=[
            pl.BlockSpec((BLK, GRAPH_DIM), lambda i: (i, 0)),
            pl.BlockSpec((2 * NUM_GRAPH, BLK), lambda i: (0, i)),
        ],
        out_shape=[
            jax.ShapeDtypeStruct((N, GRAPH_DIM), jnp.float32),
            jax.ShapeDtypeStruct((2 * NUM_GRAPH, N), jnp.float32),
        ],
    )(x, W_lin, b_lin, g8)


def _sc_body(pt_hbm, dst_hbm, src_hbm, bias_hbm, out_hbm,
             pt_v, dst_v, src_v, out_v, bias_v):
    wid = lax.axis_index("s") * NC + lax.axis_index("c")
    base = wid * EPT
    pltpu.sync_copy(pt_hbm, pt_v)
    pltpu.sync_copy(dst_hbm.at[pl.ds(base, EPT)], dst_v)
    pltpu.sync_copy(src_hbm.at[pl.ds(base, EPT)], src_v)
    pltpu.sync_copy(bias_hbm, bias_v)

    biases = [bias_v[k, :] for k in range(NUM_GRAPH)]

    def group(g, _):
        d = dst_v[pl.ds(g * L, L)]
        s = src_v[pl.ds(g * L, L)]
        obase = (g // 8) * (NUM_GRAPH * EB) + (g % 8) * L
        for k in range(NUM_GRAPH):
            a = plsc.load_gather(pt_v, [d + (k * N)])
            b = plsc.load_gather(pt_v, [s + ((NUM_GRAPH + k) * N)])
            t = a + b + biases[k]
            f = 1.0 / (1.0 + jnp.exp(-t))
            out_v[pl.ds(obase + k * EB, L)] = f
        return 0

    lax.fori_loop(0, GROUPS, group, 0)
    pltpu.sync_copy(out_v, out_hbm.at[pl.ds(base * NUM_GRAPH, EPT * NUM_GRAPH)])


@functools.partial(
    pl.kernel,
    mesh=plsc.VectorSubcoreMesh(core_axis_name="c", subcore_axis_name="s"),
    out_type=jax.ShapeDtypeStruct((EPAD * NUM_GRAPH,), jnp.float32),
    compiler_params=pltpu.CompilerParams(needs_layout_passes=False),
    scratch_types=[
        pltpu.VMEM((2 * NUM_GRAPH * N,), jnp.float32),
        pltpu.VMEM((EPT,), jnp.int32),
        pltpu.VMEM((EPT,), jnp.int32),
        pltpu.VMEM((EPT * NUM_GRAPH,), jnp.float32),
        pltpu.VMEM((NUM_GRAPH, L), jnp.float32),
    ],
)
def _sc_gate(pt_hbm, dst_hbm, src_hbm, bias_hbm, out_hbm,
             pt_v, dst_v, src_v, out_v, bias_v):
    _sc_body(pt_hbm, dst_hbm, src_hbm, bias_hbm, out_hbm,
             pt_v, dst_v, src_v, out_v, bias_v)


def kernel(x, edge_index, W_lin, b_lin, W_gate, b_gate):
    g8 = jnp.concatenate([W_gate[:, :GRAPH_DIM], W_gate[:, GRAPH_DIM:]], axis=0)
    bl = b_lin.reshape(1, GRAPH_DIM)
    bias_b = jnp.broadcast_to(b_gate[:, None], (NUM_GRAPH, L))

    src = edge_index[0]
    dst = edge_index[1]
    pad = EPAD - E
    dst_p = jnp.pad(dst, (0, pad))
    src_p = jnp.pad(src, (0, pad))

    h, pt = _tc_project(x, W_lin, bl, g8)
    out = _sc_gate(pt.reshape(-1), dst_p, src_p, bias_b)
    factors = (out.reshape(EPAD // EB, NUM_GRAPH, EB)[: E // EB]
               .transpose(0, 2, 1)
               .reshape(E, NUM_GRAPH))
    return h, factors


# speedup vs baseline: 15.1774x; 1.4014x over previous
import functools

import jax
import jax.numpy as jnp
from jax import lax
from jax.experimental import pallas as pl
from jax.experimental.pallas import tpu as pltpu
from jax.experimental.pallas import tpu_sc as plsc

N = 10000
E = 160000
IN_DIM = 256
GRAPH_DIM = 256
NUM_GRAPH = 4

BLK = 2048
NB = (N + BLK - 1) // BLK

NC, NS, L = 2, 16, 16
NW = NC * NS
EB = 128
EPT = 5120
EPAD = EPT * NW
GROUPS = EPT // L


def _tc_body(x_ref, wl_ref, bl_ref, g8_ref, h_ref, pt_ref):
    h = jnp.dot(x_ref[...], wl_ref[...], preferred_element_type=jnp.float32)
    h = h + bl_ref[...]
    h_ref[...] = h
    pt_ref[...] = lax.dot_general(
        g8_ref[...], h, (((1,), (1,)), ((), ())),
        preferred_element_type=jnp.float32)


def _tc_project(x, W_lin, b_lin, g8):
    return pl.pallas_call(
        _tc_body,
        grid=(NB,),
        in_specs=[
            pl.BlockSpec((BLK, IN_DIM), lambda i: (i, 0)),
            pl.BlockSpec((IN_DIM, GRAPH_DIM), lambda i: (0, 0)),
            pl.BlockSpec((1, GRAPH_DIM), lambda i: (0, 0)),
            pl.BlockSpec((2 * NUM_GRAPH, GRAPH_DIM), lambda i: (0, 0)),
        ],
        out_specs=[
            pl.BlockSpec((BLK, GRAPH_DIM), lambda i: (i, 0)),
            pl.BlockSpec((2 * NUM_GRAPH, BLK), lambda i: (0, i)),
        ],
        out_shape=[
            jax.ShapeDtypeStruct((N, GRAPH_DIM), jnp.float32),
            jax.ShapeDtypeStruct((2 * NUM_GRAPH, N), jnp.float32),
        ],
    )(x, W_lin, b_lin, g8)


def _sc_body(pt_hbm, dst_hbm, src_hbm, bias_hbm, out_hbm,
             pt_v, dst_v, src_v, out_v, bias_v):
    wid = lax.axis_index("s") * NC + lax.axis_index("c")
    base = wid * EPT
    pltpu.sync_copy(pt_hbm, pt_v)
    pltpu.sync_copy(dst_hbm.at[pl.ds(base, EPT)], dst_v)
    pltpu.sync_copy(src_hbm.at[pl.ds(base, EPT)], src_v)
    pltpu.sync_copy(bias_hbm, bias_v)

    biases = [bias_v[k, :] for k in range(NUM_GRAPH)]

    @plsc.parallel_loop(0, GROUPS, unroll=4)
    def group(g):
        d = dst_v[pl.ds(g * L, L)]
        s = src_v[pl.ds(g * L, L)]
        obase = (g // 8) * (NUM_GRAPH * EB) + (g % 8) * L
        for k in range(NUM_GRAPH):
            a = plsc.load_gather(pt_v, [d + (k * N)])
            b = plsc.load_gather(pt_v, [s + ((NUM_GRAPH + k) * N)])
            t = a + b + biases[k]
            f = 1.0 / (1.0 + jnp.exp(-t))
            out_v[pl.ds(obase + k * EB, L)] = f
    pltpu.sync_copy(out_v, out_hbm.at[pl.ds(base * NUM_GRAPH, EPT * NUM_GRAPH)])


@functools.partial(
    pl.kernel,
    mesh=plsc.VectorSubcoreMesh(core_axis_name="c", subcore_axis_name="s"),
    out_type=jax.ShapeDtypeStruct((EPAD * NUM_GRAPH,), jnp.float32),
    compiler_params=pltpu.CompilerParams(needs_layout_passes=False),
    scratch_types=[
        pltpu.VMEM((2 * NUM_GRAPH * N,), jnp.float32),
        pltpu.VMEM((EPT,), jnp.int32),
        pltpu.VMEM((EPT,), jnp.int32),
        pltpu.VMEM((EPT * NUM_GRAPH,), jnp.float32),
        pltpu.VMEM((NUM_GRAPH, L), jnp.float32),
    ],
)
def _sc_gate(pt_hbm, dst_hbm, src_hbm, bias_hbm, out_hbm,
             pt_v, dst_v, src_v, out_v, bias_v):
    _sc_body(pt_hbm, dst_hbm, src_hbm, bias_hbm, out_hbm,
             pt_v, dst_v, src_v, out_v, bias_v)


def kernel(x, edge_index, W_lin, b_lin, W_gate, b_gate):
    g8 = jnp.concatenate([W_gate[:, :GRAPH_DIM], W_gate[:, GRAPH_DIM:]], axis=0)
    bl = b_lin.reshape(1, GRAPH_DIM)
    bias_b = jnp.broadcast_to(b_gate[:, None], (NUM_GRAPH, L))

    src = edge_index[0]
    dst = edge_index[1]
    pad = EPAD - E
    dst_p = jnp.pad(dst, (0, pad))
    src_p = jnp.pad(src, (0, pad))

    h, pt = _tc_project(x, W_lin, bl, g8)
    out = _sc_gate(pt.reshape(-1), dst_p, src_p, bias_b)
    factors = (out.reshape(EPAD // EB, NUM_GRAPH, EB)[: E // EB]
               .transpose(0, 2, 1)
               .reshape(E, NUM_GRAPH))
    return h, factors


# speedup vs baseline: 17.9221x; 1.1808x over previous
import functools

import jax
import jax.numpy as jnp
from jax import lax
from jax.experimental import pallas as pl
from jax.experimental.pallas import tpu as pltpu
from jax.experimental.pallas import tpu_sc as plsc

N = 10000
E = 160000
IN_DIM = 256
GRAPH_DIM = 256
NUM_GRAPH = 4

BLK = 2048
NB = (N + BLK - 1) // BLK

NC, NS, L = 2, 16, 16
NW = NC * NS
EB = 128
EPT = 5120
LAST = NW - 1
EPT_LAST = E - LAST * EPT
GROUPS = EPT // L
GROUPS_LAST = EPT_LAST // L
EPAD = EPT * NW


def _tc_body(x_ref, wl_ref, bl_ref, g8_ref, h_ref, pt_ref):
    h = jnp.dot(x_ref[...], wl_ref[...], preferred_element_type=jnp.float32)
    h = h + bl_ref[...]
    h_ref[...] = h
    pt_ref[...] = lax.dot_general(
        g8_ref[...], h, (((1,), (1,)), ((), ())),
        preferred_element_type=jnp.float32)


def _tc_project(x, W_lin, b_lin, g8):
    return pl.pallas_call(
        _tc_body,
        grid=(NB,),
        in_specs=[
            pl.BlockSpec((BLK, IN_DIM), lambda i: (i, 0)),
            pl.BlockSpec((IN_DIM, GRAPH_DIM), lambda i: (0, 0)),
            pl.BlockSpec((1, GRAPH_DIM), lambda i: (0, 0)),
            pl.BlockSpec((2 * NUM_GRAPH, GRAPH_DIM), lambda i: (0, 0)),
        ],
        out_specs=[
            pl.BlockSpec((BLK, GRAPH_DIM), lambda i: (i, 0)),
            pl.BlockSpec((2 * NUM_GRAPH, BLK), lambda i: (0, i)),
        ],
        out_shape=[
            jax.ShapeDtypeStruct((N, GRAPH_DIM), jnp.float32),
            jax.ShapeDtypeStruct((2 * NUM_GRAPH, N), jnp.float32),
        ],
    )(x, W_lin, b_lin, g8)


def _sc_body(pt_hbm, edge_hbm, bias_hbm, out_hbm,
             pt_v, dst_v, src_v, out_v, bias_v):
    wid = lax.axis_index("s") * NC + lax.axis_index("c")
    base = wid * EPT
    pltpu.sync_copy(pt_hbm, pt_v)
    pltpu.sync_copy(edge_hbm.at[pl.ds(base, EPT)], src_v)
    pltpu.sync_copy(edge_hbm.at[pl.ds(E + base, EPT)], dst_v)
    pltpu.sync_copy(bias_hbm, bias_v)

    biases = [bias_v[k, :] for k in range(NUM_GRAPH)]
    n_groups = jnp.where(wid == LAST, GROUPS_LAST, GROUPS)

    @plsc.parallel_loop(0, n_groups, unroll=4)
    def group(g):
        d = dst_v[pl.ds(g * L, L)]
        s = src_v[pl.ds(g * L, L)]
        obase = (g // 8) * (NUM_GRAPH * EB) + (g % 8) * L
        for k in range(NUM_GRAPH):
            a = plsc.load_gather(pt_v, [d + (k * N)])
            b = plsc.load_gather(pt_v, [s + ((NUM_GRAPH + k) * N)])
            t = a + b + biases[k]
            f = 1.0 / (1.0 + jnp.exp(-t))
            out_v[pl.ds(obase + k * EB, L)] = f

    head = EPT_LAST * NUM_GRAPH
    pltpu.sync_copy(out_v.at[pl.ds(0, head)],
                    out_hbm.at[pl.ds(base * NUM_GRAPH, head)])

    @pl.when(wid < LAST)
    def _():
        rest = (EPT - EPT_LAST) * NUM_GRAPH
        pltpu.sync_copy(out_v.at[pl.ds(head, rest)],
                        out_hbm.at[pl.ds(base * NUM_GRAPH + head, rest)])


@functools.partial(
    pl.kernel,
    mesh=plsc.VectorSubcoreMesh(core_axis_name="c", subcore_axis_name="s"),
    out_type=jax.ShapeDtypeStruct((E * NUM_GRAPH,), jnp.float32),
    compiler_params=pltpu.CompilerParams(needs_layout_passes=False),
    scratch_types=[
        pltpu.VMEM((2 * NUM_GRAPH * N,), jnp.float32),
        pltpu.VMEM((EPT,), jnp.int32),
        pltpu.VMEM((EPT,), jnp.int32),
        pltpu.VMEM((EPT * NUM_GRAPH,), jnp.float32),
        pltpu.VMEM((NUM_GRAPH, L), jnp.float32),
    ],
)
def _sc_gate(pt_hbm, edge_hbm, bias_hbm, out_hbm,
             pt_v, dst_v, src_v, out_v, bias_v):
    _sc_body(pt_hbm, edge_hbm, bias_hbm, out_hbm,
             pt_v, dst_v, src_v, out_v, bias_v)


def kernel(x, edge_index, W_lin, b_lin, W_gate, b_gate):
    g8 = jnp.concatenate([W_gate[:, :GRAPH_DIM], W_gate[:, GRAPH_DIM:]], axis=0)
    bl = b_lin.reshape(1, GRAPH_DIM)
    bias_b = jnp.broadcast_to(b_gate[:, None], (NUM_GRAPH, L))

    edge_flat = jnp.pad(edge_index.reshape(-1), (0, EPAD - E))

    h, pt = _tc_project(x, W_lin, bl, g8)
    out = _sc_gate(pt.reshape(-1), edge_flat, bias_b)
    factors = (out.reshape(E // EB, NUM_GRAPH, EB)
               .transpose(0, 2, 1)
               .reshape(E, NUM_GRAPH))
    return h, factors
